# baseline (speedup 1.0000x reference)
import jax
import jax.numpy as jnp
from jax.experimental import pallas as pl
from jax.experimental.pallas import tpu as pltpu

B = 2
DIM = 384
MLP_DIM = 1536
HEADS = 8
DH = 48
L = 784
BH = B * HEADS
TOP_K = 128
EPS = 1e-5
HPB = 4
_F32 = jnp.float32


def _ln(x, g, b):
    m = jnp.mean(x, axis=0, keepdims=True)
    xc = x - m
    v = jnp.mean(xc * xc, axis=0, keepdims=True)
    return xc / jnp.sqrt(v + EPS) * g + b


def _attn_prep(q, k, v, ridx_row):
    qn = q / jnp.maximum(jnp.sqrt(jnp.sum(q * q, axis=0, keepdims=True)), 1e-12)
    kn = k / jnp.maximum(jnp.sqrt(jnp.sum(k * k, axis=0, keepdims=True)), 1e-12)

    key_iota = jax.lax.broadcasted_iota(jnp.int32, (L, TOP_K), 0)
    oh = (key_iota == ridx_row).astype(_F32)
    qs_t = jax.lax.dot_general(oh, qn, (((0,), (1,)), ((), ())),
                               preferred_element_type=_F32,
                               precision=jax.lax.Precision.HIGHEST)

    LP = 896
    kp = jnp.concatenate([kn, jnp.full((DH, LP - L), 1e30, _F32)], axis=1)
    mins = []
    for qb in range(4):
        acc = jnp.zeros((32, LP), _F32)
        for d in range(DH):
            acc = acc + jnp.abs(qs_t[qb * 32:(qb + 1) * 32, d:d + 1]
                                - kp[d:d + 1, :])
        mins.append(jnp.min(acc, axis=0, keepdims=True))
    min_d = jnp.minimum(jnp.minimum(mins[0], mins[1]),
                        jnp.minimum(mins[2], mins[3]))
    return qn, kn, v, min_d


def _attn_post(qn, kn, v, mask):
    maskf = mask.astype(_F32)

    csum = maskf
    sh = 1
    while sh < L:
        csum = csum + jnp.concatenate(
            [jnp.zeros((1, sh), _F32), csum[:, :L - sh]], axis=1)
        sh *= 2
    ranks = (csum - 1.0).astype(jnp.int32)
    jiota = jax.lax.broadcasted_iota(jnp.int32, (TOP_K, L), 0)
    ct = jnp.where((ranks == jiota) & mask, _F32(1), _F32(0))
    k_sel = jax.lax.dot_general(kn, ct, (((1,), (1,)), ((), ())),
                                preferred_element_type=_F32)
    v_sel = jax.lax.dot_general(v, ct, (((1,), (1,)), ((), ())),
                                preferred_element_type=_F32)

    st = jax.lax.dot_general(k_sel, qn, (((0,), (0,)), ((), ())),
                             preferred_element_type=_F32)
    mx = jnp.max(st, axis=0, keepdims=True)
    e = jnp.exp(st - mx)
    recip = 1.0 / jnp.sum(e, axis=0, keepdims=True)
    o_un = jax.lax.dot_general(v_sel, e, (((1,), (0,)), ((), ())),
                               preferred_element_type=_F32)
    return o_un * recip


def _fused_body(qs_ref, ctx_ref, cqg_ref, cqb_ref, ccg_ref, ccb_ref,
                wkv_ref, wq_ref, ridx_ref, wout_ref, cog_ref, cob_ref,
                gam_ref, w1_ref, b1_ref, g1_ref, bb1_ref,
                w2_ref, b2_ref, g2_ref, bb2_ref, out_ref, ao_ref):
    s = pl.program_id(0)
    b = s // 2
    half = s % 2

    ctxn = _ln(ctx_ref[0], ccg_ref[...], ccb_ref[...])
    qsn = _ln(qs_ref[b], cqg_ref[...], cqb_ref[...])

    rows = HPB * DH
    wk = wkv_ref[pl.ds(half * rows, rows), :]
    wv = wkv_ref[pl.ds(DIM + half * rows, rows), :]
    wq = wq_ref[pl.ds(half * rows, rows), :]
    kk = jnp.dot(wk, ctxn, preferred_element_type=_F32)
    vv = jnp.dot(wv, ctxn, preferred_element_type=_F32)
    qq = jnp.dot(wq, qsn, preferred_element_type=_F32)

    prep = []
    for hh in range(HPB):
        sl = slice(hh * DH, (hh + 1) * DH)
        prep.append(_attn_prep(qq[sl], kk[sl], vv[sl], ridx_ref[hh]))

    bits_all = jax.lax.bitcast_convert_type(
        jnp.concatenate([p[3] for p in prep], axis=0), jnp.int32)

    def bs_body(_, lohi):
        lo, hi = lohi
        mid = lo + ((hi - lo) >> 1)
        cnt = jnp.sum((bits_all <= mid).astype(jnp.int32),
                      axis=1, keepdims=True)
        pred = cnt >= TOP_K
        return (jnp.where(pred, lo, mid + 1), jnp.where(pred, mid, hi))

    _, thr = jax.lax.fori_loop(
        0, 31, bs_body,
        (jnp.zeros((HPB, 1), jnp.int32),
         jnp.full((HPB, 1), 0x7F7FFFFF, jnp.int32)))
    mask_all = bits_all <= thr

    for hh in range(HPB):
        qn, kn, v = prep[hh][:3]
        ao_ref[s * HPB + hh] = _attn_post(qn, kn, v, mask_all[hh:hh + 1, :L])

    @pl.when(s == (BH // HPB) - 1)
    def _mlp():
        g = gam_ref[...]
        bn = 1.0 / jnp.sqrt(1.0 + 1e-5)
        g1s = g1_ref[...] * bn
        w1f = w1_ref[...] * g1s
        b1f = b1_ref[...] * g1s + bb1_ref[...]
        g2s = g2_ref[...] * bn
        w2f = w2_ref[...] * g2s
        b2f = b2_ref[...] * g2s + bb2_ref[...]
        for bb in range(B):
            ain = ao_ref[bb * HEADS:(bb + 1) * HEADS].reshape(DIM, L)
            o = jnp.dot(wout_ref[...], ain, preferred_element_type=_F32)
            m = jnp.mean(o, axis=0, keepdims=True)
            oc = o - m
            var = jnp.mean(oc * oc, axis=0, keepdims=True)
            attn = (oc / jnp.sqrt(var + EPS) * (cog_ref[...] * g)
                    + cob_ref[...] * g + qs_ref[bb])
            h = jnp.dot(w1f, attn, preferred_element_type=_F32) + b1f
            h = jnp.maximum(h, 0.0)
            h2 = jnp.dot(w2f, h, preferred_element_type=_F32) + b2f
            out_ref[bb] = attn + h2


def kernel(query_source, context, cn_ctx_g, cn_ctx_b, cn_q_g, cn_q_b,
           cn_out_g, cn_out_b, W_kv, W_q, W_out, gamma, mlp_W1, mlp_b1,
           mlp_W2, mlp_b2, bn1_g, bn1_b, bn2_g, bn2_b):
    qs = query_source.reshape(B, DIM, L)
    ctx = context.reshape(B, DIM, L)
    ridx = jax.random.randint(jax.random.key(42), (BH, min(TOP_K, L)), 0, L)
    ridx = ridx.astype(jnp.int32).reshape(BH, 1, TOP_K)

    col = lambda x: x.reshape(-1, 1)
    wspec = lambda shape: pl.BlockSpec(shape, lambda s: (0,) * len(shape))

    out = pl.pallas_call(
        _fused_body,
        grid=(BH // HPB,),
        in_specs=[
            wspec((B, DIM, L)),
            pl.BlockSpec((1, DIM, L), lambda s: (s // 2, 0, 0)),
            wspec((DIM, 1)), wspec((DIM, 1)),
            wspec((DIM, 1)), wspec((DIM, 1)),
            wspec((2 * DIM, DIM)), wspec((DIM, DIM)),
            pl.BlockSpec((HPB, 1, TOP_K), lambda s: (s, 0, 0)),
            wspec((DIM, DIM)),
            wspec((DIM, 1)), wspec((DIM, 1)), wspec((1, 1)),
            wspec((MLP_DIM, DIM)), wspec((MLP_DIM, 1)),
            wspec((MLP_DIM, 1)), wspec((MLP_DIM, 1)),
            wspec((DIM, MLP_DIM)), wspec((DIM, 1)),
            wspec((DIM, 1)), wspec((DIM, 1)),
        ],
        out_specs=pl.BlockSpec((B, DIM, L), lambda s: (0, 0, 0)),
        out_shape=jax.ShapeDtypeStruct((B, DIM, L), _F32),
        scratch_shapes=[pltpu.VMEM((BH, DH, L), _F32)],
    )(qs, ctx, col(cn_q_g), col(cn_q_b), col(cn_ctx_g), col(cn_ctx_b),
      W_kv, W_q, ridx, W_out, col(cn_out_g), col(cn_out_b),
      gamma.reshape(1, 1), mlp_W1, col(mlp_b1), col(bn1_g), col(bn1_b),
      mlp_W2, col(mlp_b2), col(bn2_g), col(bn2_b))

    return out.reshape(query_source.shape)


# speedup vs baseline: 1.0709x; 1.0709x over previous
import jax
import jax.numpy as jnp
from jax.experimental import pallas as pl

B = 2
DIM = 384
MLP_DIM = 1536
HEADS = 8
DH = 48
L = 784
BH = B * HEADS
TOP_K = 128
EPS = 1e-5
_F32 = jnp.float32


def _proj_body(qs_ref, ctx_ref, cqg_ref, cqb_ref, ccg_ref, ccb_ref,
               wk_ref, wv_ref, wq_ref, k_ref, v_ref, q_ref):
    def ln(x, g, b):
        m = jnp.mean(x, axis=0, keepdims=True)
        xc = x - m
        v = jnp.mean(xc * xc, axis=0, keepdims=True)
        return xc / jnp.sqrt(v + EPS) * g + b

    ctxn = ln(ctx_ref[0], ccg_ref[...], ccb_ref[...])
    qsn = ln(qs_ref[0], cqg_ref[...], cqb_ref[...])
    k_ref[...] = jnp.dot(wk_ref[...], ctxn,
                         preferred_element_type=_F32).reshape(HEADS, DH, L)
    v_ref[...] = jnp.dot(wv_ref[...], ctxn,
                         preferred_element_type=_F32).reshape(HEADS, DH, L)
    q_ref[...] = jnp.dot(wq_ref[...], qsn,
                         preferred_element_type=_F32).reshape(HEADS, DH, L)


HPB = 8


def _attn_body(q_ref, k_ref, v_ref, ridx_ref, o_ref):
    prep = []
    for hh in range(HPB):
        prep.append(_attn_prep(q_ref[hh], k_ref[hh], v_ref[hh], ridx_ref[hh]))

    bits_all = jax.lax.bitcast_convert_type(
        jnp.concatenate([p[3] for p in prep], axis=0), jnp.int32)

    def bs_body(_, lohi):
        lo, hi = lohi
        mid = lo + ((hi - lo) >> 1)
        cnt = jnp.sum((bits_all <= mid).astype(jnp.int32),
                      axis=1, keepdims=True)
        pred = cnt >= TOP_K
        return (jnp.where(pred, lo, mid + 1), jnp.where(pred, mid, hi))

    _, thr = jax.lax.fori_loop(
        0, 31, bs_body,
        (jnp.zeros((HPB, 1), jnp.int32), jnp.full((HPB, 1), 0x7F7FFFFF,
                                                  jnp.int32)))
    mask_all = bits_all <= thr

    for hh in range(HPB):
        qn, kn, v = prep[hh][:3]
        o_ref[hh] = _attn_post(qn, kn, v, mask_all[hh:hh + 1, :L])


def _attn_prep(q, k, v, ridx_row):
    qn = q / jnp.maximum(jnp.sqrt(jnp.sum(q * q, axis=0, keepdims=True)), 1e-12)
    kn = k / jnp.maximum(jnp.sqrt(jnp.sum(k * k, axis=0, keepdims=True)), 1e-12)

    ridx = ridx_row
    key_iota = jax.lax.broadcasted_iota(jnp.int32, (L, TOP_K), 0)
    oh = (key_iota == ridx).astype(_F32)
    qs_t = jax.lax.dot_general(oh, qn, (((0,), (1,)), ((), ())),
                               preferred_element_type=_F32,
                               precision=jax.lax.Precision.HIGHEST)

    LP = 896
    kp = jnp.concatenate([kn, jnp.full((DH, LP - L), 1e30, _F32)], axis=1)
    mins = []
    for qb in range(4):
        acc = jnp.zeros((32, LP), _F32)
        for d in range(DH):
            acc = acc + jnp.abs(qs_t[qb * 32:(qb + 1) * 32, d:d + 1]
                                - kp[d:d + 1, :])
        mins.append(jnp.min(acc, axis=0, keepdims=True))
    min_d = jnp.minimum(jnp.minimum(mins[0], mins[1]),
                        jnp.minimum(mins[2], mins[3]))
    return qn, kn, v, min_d


def _attn_post(qn, kn, v, mask):
    maskf = mask.astype(_F32)

    csum = maskf
    sh = 1
    while sh < L:
        csum = csum + jnp.concatenate(
            [jnp.zeros((1, sh), _F32), csum[:, :L - sh]], axis=1)
        sh *= 2
    ranks = (csum - 1.0).astype(jnp.int32)
    jiota = jax.lax.broadcasted_iota(jnp.int32, (TOP_K, L), 0)
    ct = jnp.where((ranks == jiota) & mask, _F32(1), _F32(0))
    k_sel = jax.lax.dot_general(kn, ct, (((1,), (1,)), ((), ())),
                                preferred_element_type=_F32)
    v_sel = jax.lax.dot_general(v, ct, (((1,), (1,)), ((), ())),
                                preferred_element_type=_F32)

    st = jax.lax.dot_general(k_sel, qn, (((0,), (0,)), ((), ())),
                             preferred_element_type=_F32)
    mx = jnp.max(st, axis=0, keepdims=True)
    e = jnp.exp(st - mx)
    recip = 1.0 / jnp.sum(e, axis=0, keepdims=True)
    o_un = jax.lax.dot_general(v_sel, e, (((1,), (0,)), ((), ())),
                               preferred_element_type=_F32)
    return o_un * recip


def _mlp_body(a_ref, qs_ref, wout_ref, cog_ref, cob_ref, gam_ref,
              w1_ref, b1_ref, g1_ref, bb1_ref,
              w2_ref, b2_ref, g2_ref, bb2_ref, out_ref):
    o = jnp.dot(wout_ref[...], a_ref[0], preferred_element_type=_F32)
    m = jnp.mean(o, axis=0, keepdims=True)
    oc = o - m
    var = jnp.mean(oc * oc, axis=0, keepdims=True)
    g = gam_ref[...]
    attn = (oc / jnp.sqrt(var + EPS) * (cog_ref[...] * g)
            + cob_ref[...] * g + qs_ref[0])
    s = 1.0 / jnp.sqrt(1.0 + EPS)
    g1s = g1_ref[...] * s
    w1f = w1_ref[...] * g1s
    b1f = b1_ref[...] * g1s + bb1_ref[...]
    h = jnp.dot(w1f, attn, preferred_element_type=_F32) + b1f
    h = jnp.maximum(h, 0.0)
    g2s = g2_ref[...] * s
    w2f = w2_ref[...] * g2s
    b2f = b2_ref[...] * g2s + bb2_ref[...]
    h2 = jnp.dot(w2f, h, preferred_element_type=_F32) + b2f
    out_ref[0] = attn + h2


def kernel(query_source, context, cn_ctx_g, cn_ctx_b, cn_q_g, cn_q_b,
           cn_out_g, cn_out_b, W_kv, W_q, W_out, gamma, mlp_W1, mlp_b1,
           mlp_W2, mlp_b2, bn1_g, bn1_b, bn2_g, bn2_b):
    qs = query_source.reshape(B, DIM, L)
    ctx = context.reshape(B, DIM, L)
    ridx = jax.random.randint(jax.random.key(42), (BH, min(TOP_K, L)), 0, L)
    ridx = ridx.astype(jnp.int32).reshape(BH, 1, TOP_K)

    col = lambda x: x.reshape(-1, 1)
    wspec = lambda shape: pl.BlockSpec(shape, lambda b: (0,) * len(shape))

    kh, vh, qh = pl.pallas_call(
        _proj_body,
        grid=(B,),
        in_specs=[
            pl.BlockSpec((1, DIM, L), lambda b: (b, 0, 0)),
            pl.BlockSpec((1, DIM, L), lambda b: (b, 0, 0)),
            wspec((DIM, 1)), wspec((DIM, 1)), wspec((DIM, 1)), wspec((DIM, 1)),
            wspec((DIM, DIM)), wspec((DIM, DIM)), wspec((DIM, DIM)),
        ],
        out_specs=[
            pl.BlockSpec((HEADS, DH, L), lambda b: (b, 0, 0)),
            pl.BlockSpec((HEADS, DH, L), lambda b: (b, 0, 0)),
            pl.BlockSpec((HEADS, DH, L), lambda b: (b, 0, 0)),
        ],
        out_shape=[
            jax.ShapeDtypeStruct((BH, DH, L), _F32),
            jax.ShapeDtypeStruct((BH, DH, L), _F32),
            jax.ShapeDtypeStruct((BH, DH, L), _F32),
        ],
    )(qs, ctx, col(cn_q_g), col(cn_q_b), col(cn_ctx_g), col(cn_ctx_b),
      W_kv[:DIM], W_kv[DIM:], W_q)

    ao = pl.pallas_call(
        _attn_body,
        grid=(BH // HPB,),
        in_specs=[
            pl.BlockSpec((HPB, DH, L), lambda i: (i, 0, 0)),
            pl.BlockSpec((HPB, DH, L), lambda i: (i, 0, 0)),
            pl.BlockSpec((HPB, DH, L), lambda i: (i, 0, 0)),
            pl.BlockSpec((HPB, 1, TOP_K), lambda i: (i, 0, 0)),
        ],
        out_specs=pl.BlockSpec((HPB, DH, L), lambda i: (i, 0, 0)),
        out_shape=jax.ShapeDtypeStruct((BH, DH, L), _F32),
    )(qh, kh, vh, ridx)

    attn_in = ao.reshape(B, DIM, L)

    out = pl.pallas_call(
        _mlp_body,
        grid=(B,),
        in_specs=[
            pl.BlockSpec((1, DIM, L), lambda b: (b, 0, 0)),
            pl.BlockSpec((1, DIM, L), lambda b: (b, 0, 0)),
            wspec((DIM, DIM)),
            wspec((DIM, 1)), wspec((DIM, 1)), wspec((1, 1)),
            wspec((MLP_DIM, DIM)), wspec((MLP_DIM, 1)),
            wspec((MLP_DIM, 1)), wspec((MLP_DIM, 1)),
            wspec((DIM, MLP_DIM)), wspec((DIM, 1)),
            wspec((DIM, 1)), wspec((DIM, 1)),
        ],
        out_specs=pl.BlockSpec((1, DIM, L), lambda b: (b, 0, 0)),
        out_shape=jax.ShapeDtypeStruct((B, DIM, L), _F32),
    )(attn_in, qs, W_out, col(cn_out_g), col(cn_out_b), gamma.reshape(1, 1),
      mlp_W1, col(mlp_b1), col(bn1_g), col(bn1_b),
      mlp_W2, col(mlp_b2), col(bn2_g), col(bn2_b))

    return out.reshape(query_source.shape)
